# baseline (speedup 1.0000x reference)
import jax
import jax.numpy as jnp
from jax import lax
from jax.experimental import pallas as pl
from jax.experimental.pallas import tpu as pltpu
from jax.experimental.pallas import tpu_sc as plsc

MAX_LEN = 8192
EMB_DIM = 64
PAD_DIM = 128
N_ROWS = 4096
N_COLS = 200
L = 16

NC = 2
NS = 16
NW = NC * NS
BLK = N_ROWS // NW


def _body(idx_hbm, table_hbm, out_hbm,
          idx_v, g0, g1, g2, g3, t0, t1,
          sg0, sg1, sg2, sg3, sw0, sw1):
    wid = lax.axis_index("s") * NC + lax.axis_index("c")

    pltpu.sync_copy(idx_hbm.at[wid], idx_v)

    g_bufs = (g0, g1, g2, g3)
    t_bufs = (t0, t1)
    g_sems = (sg0, sg1, sg2, sg3)
    w_sems = (sw0, sw1)

    iota = lax.iota(jnp.int32, L)
    diag = [(iota + k) & (L - 1) for k in range(L)]
    l_base = [iota + L * lb for lb in range(BLK // L)]

    def gather(j, p):
        pltpu.async_copy(table_hbm.at[idx_v.at[j]], g_bufs[p], g_sems[p])

    def gather_wait(p):
        pltpu.make_async_copy(table_hbm.at[idx_v.at[0]], g_bufs[p], g_sems[p]).wait()

    def transpose(p, tp):
        g, t = g_bufs[p], t_bufs[tp]

        def cblock2(h, carry):
            for cb2 in range(2):
                c0 = h * (2 * L) + cb2 * L
                for lb in range(BLK // L):
                    rows = l_base[lb]
                    for k in range(L):
                        cols = diag[k] + c0
                        v = plsc.load_gather(g, [rows, cols])
                        plsc.store_scatter(t, [cols, rows], v)
            return carry

        lax.fori_loop(0, EMB_DIM // (2 * L), cblock2, 0)

    def write(j, p):
        pltpu.async_copy(t_bufs[p], out_hbm.at[j, :, pl.ds(wid * BLK, BLK)], w_sems[p])

    def write_wait(p):
        pltpu.make_async_copy(t_bufs[p], out_hbm.at[0, :, pl.ds(0, BLK)], w_sems[p]).wait()

    for p in range(4):
        gather(p, p)

    def step(jj, carry):
        for p in range(4):
            j = jj * 4 + p
            tp = p & 1
            gather_wait(p)

            @pl.when(j >= 2)
            def _():
                write_wait(tp)

            transpose(p, tp)
            write(j, tp)

            @pl.when(j + 4 < N_COLS)
            def _():
                gather(j + 4, p)

        return carry

    lax.fori_loop(0, N_COLS // 4, step, 0)
    write_wait(0)
    write_wait(1)


@jax.jit
def _gather_op(positions, table_padded):
    mesh = plsc.VectorSubcoreMesh(core_axis_name="c", subcore_axis_name="s")
    idx = positions.reshape(NW, BLK, N_COLS).transpose(0, 2, 1)
    out = pl.kernel(
        _body,
        out_type=jax.ShapeDtypeStruct((N_COLS, EMB_DIM, N_ROWS), jnp.float32),
        mesh=mesh,
        scratch_types=[
            pltpu.VMEM((N_COLS, BLK), jnp.int32),
            pltpu.VMEM((BLK, PAD_DIM), jnp.float32),
            pltpu.VMEM((BLK, PAD_DIM), jnp.float32),
            pltpu.VMEM((BLK, PAD_DIM), jnp.float32),
            pltpu.VMEM((BLK, PAD_DIM), jnp.float32),
            pltpu.VMEM((EMB_DIM, BLK), jnp.float32),
            pltpu.VMEM((EMB_DIM, BLK), jnp.float32),
            pltpu.SemaphoreType.DMA,
            pltpu.SemaphoreType.DMA,
            pltpu.SemaphoreType.DMA,
            pltpu.SemaphoreType.DMA,
            pltpu.SemaphoreType.DMA,
            pltpu.SemaphoreType.DMA,
        ],
        compiler_params=pltpu.CompilerParams(
            use_tc_tiling_on_sc=True, needs_layout_passes=False),
    )(idx, table_padded)
    return jnp.transpose(out, (2, 0, 1))


def kernel(positions, table):
    table_padded = jnp.pad(table, ((0, 0), (0, PAD_DIM - EMB_DIM)))
    return _gather_op(positions, table_padded)


# speedup vs baseline: 1.2363x; 1.2363x over previous
import jax
import jax.numpy as jnp
from jax import lax
from jax.experimental import pallas as pl
from jax.experimental.pallas import tpu as pltpu
from jax.experimental.pallas import tpu_sc as plsc

MAX_LEN = 8192
EMB_DIM = 64
PAD_DIM = 128
N_ROWS = 4096
N_COLS = 200
L = 16

NC = 2
NS = 16
NW = NC * NS
BLK = N_ROWS // NW


def _body(idx_hbm, table_hbm, out_hbm,
          idx_v, g0, g1, g2, g3, t0, t1,
          sg0, sg1, sg2, sg3, sw0, sw1):
    wid = lax.axis_index("s") * NC + lax.axis_index("c")

    pltpu.sync_copy(idx_hbm.at[wid], idx_v)

    g_bufs = (g0, g1, g2, g3)
    t_bufs = (t0, t1)
    g_sems = (sg0, sg1, sg2, sg3)
    w_sems = (sw0, sw1)

    iota = lax.iota(jnp.int32, L)
    diag = [(iota + k) & (L - 1) for k in range(L)]
    l_base = [iota + L * lb for lb in range(BLK // L)]

    def gather(j, p):
        pltpu.async_copy(table_hbm.at[idx_v.at[j]], g_bufs[p], g_sems[p])

    def gather_wait(p):
        pltpu.make_async_copy(table_hbm.at[idx_v.at[0]], g_bufs[p], g_sems[p]).wait()

    def transpose(p, tp):
        g, t = g_bufs[p], t_bufs[tp]

        def cblock(cb, carry):
            c0 = cb * L
            for lb in range(BLK // L):
                rows = l_base[lb]
                for k in range(L):
                    cols = diag[k] + c0
                    v = plsc.load_gather(g, [rows, cols])
                    plsc.store_scatter(t, [cols, rows], v)
            return carry

        lax.fori_loop(0, EMB_DIM // L, cblock, 0)

    def write(j, p):
        pltpu.async_copy(t_bufs[p], out_hbm.at[j, :, pl.ds(wid * BLK, BLK)], w_sems[p])

    def write_wait(p):
        pltpu.make_async_copy(t_bufs[p], out_hbm.at[0, :, pl.ds(0, BLK)], w_sems[p]).wait()

    for p in range(4):
        gather(p, p)

    def step(jj, carry):
        for p in range(4):
            j = jj * 4 + p
            tp = p & 1
            gather_wait(p)

            @pl.when(j >= 2)
            def _():
                write_wait(tp)

            transpose(p, tp)
            write(j, tp)

            @pl.when(j + 4 < N_COLS)
            def _():
                gather(j + 4, p)

        return carry

    lax.fori_loop(0, N_COLS // 4, step, 0)
    write_wait(0)
    write_wait(1)


@jax.jit
def _gather_op(positions, table_padded):
    mesh = plsc.VectorSubcoreMesh(core_axis_name="c", subcore_axis_name="s")
    idx = positions.reshape(NW, BLK, N_COLS).transpose(0, 2, 1)
    out = pl.kernel(
        _body,
        out_type=jax.ShapeDtypeStruct((N_COLS, EMB_DIM, N_ROWS), jnp.float32),
        mesh=mesh,
        scratch_types=[
            pltpu.VMEM((N_COLS, BLK), jnp.int32),
            pltpu.VMEM((BLK, PAD_DIM), jnp.float32),
            pltpu.VMEM((BLK, PAD_DIM), jnp.float32),
            pltpu.VMEM((BLK, PAD_DIM), jnp.float32),
            pltpu.VMEM((BLK, PAD_DIM), jnp.float32),
            pltpu.VMEM((EMB_DIM, BLK), jnp.float32),
            pltpu.VMEM((EMB_DIM, BLK), jnp.float32),
            pltpu.SemaphoreType.DMA,
            pltpu.SemaphoreType.DMA,
            pltpu.SemaphoreType.DMA,
            pltpu.SemaphoreType.DMA,
            pltpu.SemaphoreType.DMA,
            pltpu.SemaphoreType.DMA,
        ],
        compiler_params=pltpu.CompilerParams(
            use_tc_tiling_on_sc=True, needs_layout_passes=False),
    )(idx, table_padded)
    return jnp.transpose(out, (2, 0, 1))


def kernel(positions, table):
    table_padded = jnp.pad(table, ((0, 0), (0, PAD_DIM - EMB_DIM)))
    return _gather_op(positions, table_padded)


# speedup vs baseline: 2.2032x; 1.7820x over previous
import jax
import jax.numpy as jnp
from jax import lax
from jax.experimental import pallas as pl
from jax.experimental.pallas import tpu as pltpu
from jax.experimental.pallas import tpu_sc as plsc

MAX_LEN = 8192
EMB_DIM = 64
PAD_DIM = 128
N_ROWS = 4096
N_COLS = 200
L = 16

NC = 2
NS = 16
NW = NC * NS
BLK = N_ROWS // NW


def _body(idx_hbm, table_hbm, out_hbm,
          idx_v, g0, g1, g2, g3, t0, t1,
          sg0, sg1, sg2, sg3, sw0, sw1):
    wid = lax.axis_index("s") * NC + lax.axis_index("c")

    pltpu.sync_copy(idx_hbm.at[wid], idx_v)

    g_bufs = (g0, g1, g2, g3)
    t_bufs = (t0, t1)
    g_sems = (sg0, sg1, sg2, sg3)
    w_sems = (sw0, sw1)

    iota = lax.iota(jnp.int32, L)
    diag = [(iota + k) & (L - 1) for k in range(L)]
    l_base = [iota + L * lb for lb in range(BLK // L)]

    def gather(j, p):
        pltpu.async_copy(table_hbm.at[idx_v.at[j]], g_bufs[p], g_sems[p])

    def gather_wait(p):
        pltpu.make_async_copy(table_hbm.at[idx_v.at[0]], g_bufs[p], g_sems[p]).wait()

    def transpose(p, tp):
        g, t = g_bufs[p], t_bufs[tp]

        def cblock(cb, carry):
            c0 = cb * L
            for lb in range(BLK // L):
                rows = l_base[lb]
                for kh in range(2):
                    batch = []
                    for k in range(kh * 8, kh * 8 + 8):
                        cols = diag[k] + c0
                        batch.append((cols, plsc.load_gather(g, [rows, cols])))
                    for cols, v in batch:
                        plsc.store_scatter(t, [cols, rows], v)
            return carry

        lax.fori_loop(0, EMB_DIM // L, cblock, 0)

    def write(j, p):
        pltpu.async_copy(t_bufs[p], out_hbm.at[j, :, pl.ds(wid * BLK, BLK)], w_sems[p])

    def write_wait(p):
        pltpu.make_async_copy(t_bufs[p], out_hbm.at[0, :, pl.ds(0, BLK)], w_sems[p]).wait()

    for p in range(4):
        gather(p, p)

    def step(jj, carry):
        for p in range(4):
            j = jj * 4 + p
            tp = p & 1
            gather_wait(p)

            @pl.when(j >= 2)
            def _():
                write_wait(tp)

            transpose(p, tp)
            write(j, tp)

            @pl.when(j + 4 < N_COLS)
            def _():
                gather(j + 4, p)

        return carry

    lax.fori_loop(0, N_COLS // 4, step, 0)
    write_wait(0)
    write_wait(1)


@jax.jit
def _gather_op(positions, table_padded):
    mesh = plsc.VectorSubcoreMesh(core_axis_name="c", subcore_axis_name="s")
    idx = positions.reshape(NW, BLK, N_COLS).transpose(0, 2, 1)
    out = pl.kernel(
        _body,
        out_type=jax.ShapeDtypeStruct((N_COLS, EMB_DIM, N_ROWS), jnp.float32),
        mesh=mesh,
        scratch_types=[
            pltpu.VMEM((N_COLS, BLK), jnp.int32),
            pltpu.VMEM((BLK, PAD_DIM), jnp.float32),
            pltpu.VMEM((BLK, PAD_DIM), jnp.float32),
            pltpu.VMEM((BLK, PAD_DIM), jnp.float32),
            pltpu.VMEM((BLK, PAD_DIM), jnp.float32),
            pltpu.VMEM((EMB_DIM, BLK), jnp.float32),
            pltpu.VMEM((EMB_DIM, BLK), jnp.float32),
            pltpu.SemaphoreType.DMA,
            pltpu.SemaphoreType.DMA,
            pltpu.SemaphoreType.DMA,
            pltpu.SemaphoreType.DMA,
            pltpu.SemaphoreType.DMA,
            pltpu.SemaphoreType.DMA,
        ],
        compiler_params=pltpu.CompilerParams(
            use_tc_tiling_on_sc=True, needs_layout_passes=False),
    )(idx, table_padded)
    return jnp.transpose(out, (2, 0, 1))


def kernel(positions, table):
    table_padded = jnp.pad(table, ((0, 0), (0, PAD_DIM - EMB_DIM)))
    return _gather_op(positions, table_padded)
